# baseline (speedup 1.0000x reference)
import jax
import jax.numpy as jnp
from jax.experimental import pallas as pl
from jax.experimental.pallas import tpu as pltpu

N, K, T, H = 32, 16, 128, 1024
NEGINF = -1e20


def _body(ids_ref, qcat_ref, wcqk_t_ref, bcqk_ref, wk_ref, bk_ref, ckm_ref,
          pool1_full_ref, pool0_ref, pool1_sel_ref, mask_ref, pidx_ref,
          score_ref, enc_ref, use_ref, mask_out_ref, pidx_out_ref):
    n = pl.program_id(0)
    enc_ref[...] = pool0_ref[...]
    use_ref[...] = pool1_sel_ref[...]
    mask_out_ref[...] = mask_ref[...]
    pidx_out_ref[...] = pidx_ref[...]

    @pl.when(n == 0)
    def _():
        cq = jnp.dot(qcat_ref[...], wcqk_t_ref[...],
                     precision=jax.lax.Precision.HIGHEST) + bcqk_ref[...]
        t = jnp.dot(cq, wk_ref[...],
                    precision=jax.lax.Precision.HIGHEST)
        bias = jnp.sum(cq * bk_ref[...], axis=1, keepdims=True)
        s = jnp.sum(pool1_full_ref[...] * t[:, None, :], axis=2) + bias
        score_ref[...] = jnp.where(ckm_ref[...] != 0, s, NEGINF)


def kernel(contexts_encoded_1, tracked_knowledge_use,
           knowledge_shifting_pool_encoded_0, knowledge_shifting_pool_encoded_1,
           knowledge_shifting_pool_mask, shifting_ck_mask,
           knowledge_shifting_label, knowledge_shifting_pool,
           W_cqk, b_cqk, W_k, b_k):
    ids = knowledge_shifting_label.astype(jnp.int32)
    qcat = jnp.concatenate(
        [contexts_encoded_1[:, 2, :], tracked_knowledge_use], axis=1)
    pool0 = knowledge_shifting_pool_encoded_0.reshape(N * K, T, H)
    pool1_sel = knowledge_shifting_pool_encoded_1.reshape(N * K, 1, H)
    mask_i32 = knowledge_shifting_pool_mask.astype(jnp.int32).reshape(N * K, 1, T)
    pidx = knowledge_shifting_pool.reshape(N * K, 1, T)
    ckm = shifting_ck_mask.astype(jnp.int32)

    def sel(nb):
        def im(n, s):
            return (n * K + s[n],) + (0,) * (nb - 1)
        return im

    grid_spec = pltpu.PrefetchScalarGridSpec(
        num_scalar_prefetch=1,
        grid=(N,),
        in_specs=[
            pl.BlockSpec((N, 2 * H), lambda n, s: (0, 0)),
            pl.BlockSpec((2 * H, H), lambda n, s: (0, 0)),
            pl.BlockSpec((1, H), lambda n, s: (0, 0)),
            pl.BlockSpec((H, H), lambda n, s: (0, 0)),
            pl.BlockSpec((1, H), lambda n, s: (0, 0)),
            pl.BlockSpec((N, K), lambda n, s: (0, 0)),
            pl.BlockSpec((N, K, H), lambda n, s: (0, 0, 0)),
            pl.BlockSpec((1, T, H), sel(3)),
            pl.BlockSpec((1, 1, H), sel(3)),
            pl.BlockSpec((1, 1, T), sel(3)),
            pl.BlockSpec((1, 1, T), sel(3)),
        ],
        out_specs=[
            pl.BlockSpec((N, K), lambda n, s: (0, 0)),
            pl.BlockSpec((1, T, H), lambda n, s: (n, 0, 0)),
            pl.BlockSpec((1, 1, H), lambda n, s: (n, 0, 0)),
            pl.BlockSpec((1, 1, T), lambda n, s: (n, 0, 0)),
            pl.BlockSpec((1, 1, T), lambda n, s: (n, 0, 0)),
        ],
    )

    score, enc, use, mask_out, pidx_out = pl.pallas_call(
        _body,
        grid_spec=grid_spec,
        out_shape=[
            jax.ShapeDtypeStruct((N, K), jnp.float32),
            jax.ShapeDtypeStruct((N, T, H), jnp.float32),
            jax.ShapeDtypeStruct((N, 1, H), jnp.float32),
            jax.ShapeDtypeStruct((N, 1, T), jnp.int32),
            jax.ShapeDtypeStruct((N, 1, T), jnp.int32),
        ],
    )(ids, qcat, W_cqk.T, b_cqk.reshape(1, H), W_k, b_k.reshape(1, H), ckm,
      knowledge_shifting_pool_encoded_1, pool0, pool1_sel, mask_i32, pidx)

    return (score, enc,
            mask_out.reshape(N, T).astype(bool),
            use.reshape(N, H),
            pidx_out.reshape(N, T).astype(knowledge_shifting_pool.dtype))
